# baseline (speedup 1.0000x reference)
import functools

import jax
import jax.numpy as jnp
import numpy as np
from jax import lax
from jax.experimental import pallas as pl
from jax.experimental.pallas import tpu as pltpu
from jax.experimental.pallas import tpu_sc as plsc

_K = 2
_CAPF = 1.25

_NC = 2
_NS = 16
_NW = _NC * _NS
_L = 16

_TILE = 256


def _router_body(cap, T, E, x_ref, wg_ref, cs0_ref, cs1_ref, ds0_ref, ds1_ref,
                 w0_ref, w1_ref, cnt_ref):
    logits = jnp.dot(x_ref[...], wg_ref[...], preferred_element_type=jnp.float32)
    iota_e = lax.broadcasted_iota(jnp.int32, (T, E), 1)
    m = jnp.max(logits, axis=1, keepdims=True)
    p = jnp.exp(logits - m)
    p1 = jnp.max(p, axis=1, keepdims=True)
    i1 = jnp.min(jnp.where(p == p1, iota_e, E), axis=1, keepdims=True)
    pm = jnp.where(iota_e == i1, -1.0, p)
    p2 = jnp.max(pm, axis=1, keepdims=True)
    i2 = jnp.min(jnp.where(pm == p2, iota_e, E), axis=1, keepdims=True)
    denom = p1 + p2
    w0 = p1 / denom
    w1 = p2 / denom

    oh2 = ((iota_e == i1) | (iota_e == i2)).astype(jnp.int32)
    c = oh2
    sh = 1
    while sh < T:
        c = c + jnp.concatenate(
            [jnp.zeros((sh, E), jnp.int32), c[:-sh]], axis=0)
        sh *= 2
    excl = c - oh2

    pos0 = jnp.sum(jnp.where(iota_e == i1, excl, 0), axis=1, keepdims=True)
    pos1 = jnp.sum(jnp.where(iota_e == i2, excl, 0), axis=1, keepdims=True)
    keep0 = pos0 < cap
    keep1 = pos1 < cap
    slot0 = i1 * cap + pos0
    slot1 = i2 * cap + pos1
    trash = E * cap

    cs0_ref[...] = jnp.broadcast_to(jnp.where(keep0, slot0, 0), (T, E))
    cs1_ref[...] = jnp.broadcast_to(jnp.where(keep1, slot1, 0), (T, E))
    ds0_ref[...] = jnp.broadcast_to(jnp.where(keep0, slot0, trash), (T, E))
    ds1_ref[...] = jnp.broadcast_to(jnp.where(keep1, slot1, trash), (T, E))
    w0_ref[...] = jnp.broadcast_to(jnp.where(keep0, w0, 0.0), (T, _L))
    w1_ref[...] = jnp.broadcast_to(jnp.where(keep1, w1, 0.0), (T, _L))
    counts = c[T - 1:T, :]
    cnt_ref[...] = jnp.minimum(counts, cap)


def _expert_body(cap, nt, cnt_ref, ein_ref, w1_ref, b1_ref, w2_ref, b2_ref,
                 out_ref):
    i = pl.program_id(0)
    e = i // nt
    tile_start = (i % nt) * _TILE
    nvalid = cnt_ref[0, e] - tile_start

    @pl.when(nvalid > 0)
    def _compute():
        xt = ein_ref[...].astype(jnp.bfloat16)
        h = jnp.dot(xt, w1_ref[0].astype(jnp.bfloat16),
                    preferred_element_type=jnp.float32)
        h = jax.nn.gelu(h + b1_ref[0]).astype(jnp.bfloat16)
        o = jnp.dot(h, w2_ref[0].astype(jnp.bfloat16),
                    preferred_element_type=jnp.float32)
        o = o + b2_ref[0]
        rows = lax.broadcasted_iota(jnp.int32, (_TILE, 1), 0)
        out_ref[...] = jnp.where(rows < nvalid, o, 0.0)

    @pl.when(nvalid <= 0)
    def _zero():
        out_ref[...] = jnp.zeros_like(out_ref)


def _dispatch_body(T, D, sub, nsub, xf_hbm, d0_hbm, d1_hbm, ein_hbm,
                   xbuf, idx0, idx1, sem0, sem1):
    wid = lax.axis_index("s") * _NC + lax.axis_index("c")
    tpw = T // _NW

    def body(s, carry):
        base = wid * tpw + s * sub
        pltpu.sync_copy(d0_hbm.at[pl.ds(base, sub)], idx0)
        pltpu.sync_copy(d1_hbm.at[pl.ds(base, sub)], idx1)
        pltpu.sync_copy(xf_hbm.at[pl.ds(base, sub)], xbuf)
        cp0 = pltpu.async_copy(xbuf, ein_hbm.at[idx0], sem0)
        cp1 = pltpu.async_copy(xbuf, ein_hbm.at[idx1], sem1)
        cp0.wait()
        cp1.wait()
        return carry

    lax.fori_loop(0, nsub, body, 0)


def _combine_body(T, D, sub, nsub, eout_hbm, c0_hbm, c1_hbm, w0_hbm, w1_hbm,
                  y_hbm, r0, r1, idx0, idx1, w0v, w1v, sem0, sem1):
    wid = lax.axis_index("s") * _NC + lax.axis_index("c")
    tpw = T // _NW
    nd = D // _L

    def body(s, carry):
        base = wid * tpw + s * sub
        pltpu.sync_copy(c0_hbm.at[pl.ds(base, sub)], idx0)
        pltpu.sync_copy(c1_hbm.at[pl.ds(base, sub)], idx1)
        pltpu.sync_copy(w0_hbm.at[pl.ds(base, sub)], w0v)
        pltpu.sync_copy(w1_hbm.at[pl.ds(base, sub)], w1v)
        cp0 = pltpu.async_copy(eout_hbm.at[idx0], r0, sem0)
        cp1 = pltpu.async_copy(eout_hbm.at[idx1], r1, sem1)
        cp0.wait()
        cp1.wait()

        def tok_body(i, tc):
            a = w0v[i, pl.ds(0, _L)]
            b = w1v[i, pl.ds(0, _L)]

            def d_body(d, dc):
                for u in range(4):
                    off = d * (4 * _L) + u * _L
                    v = a * r0[i, pl.ds(off, _L)] + b * r1[i, pl.ds(off, _L)]
                    r0[i, pl.ds(off, _L)] = v
                return dc

            lax.fori_loop(0, nd // 4, d_body, 0)
            return tc

        lax.fori_loop(0, sub, tok_body, 0)
        pltpu.sync_copy(r0, y_hbm.at[pl.ds(base, sub)])
        return carry

    lax.fori_loop(0, nsub, body, 0)


def kernel(x, Wg, W1, b1, W2, b2):
    Bx, Sx, D = x.shape
    T = Bx * Sx
    E = Wg.shape[1]
    H = W1.shape[2]
    cap = int(np.ceil(T * _K / E * _CAPF))
    nt = cap // _TILE
    xf = x.reshape(T, D)

    router = pl.pallas_call(
        functools.partial(_router_body, cap, T, E),
        out_shape=(
            jax.ShapeDtypeStruct((T, E), jnp.int32),
            jax.ShapeDtypeStruct((T, E), jnp.int32),
            jax.ShapeDtypeStruct((T, E), jnp.int32),
            jax.ShapeDtypeStruct((T, E), jnp.int32),
            jax.ShapeDtypeStruct((T, _L), jnp.float32),
            jax.ShapeDtypeStruct((T, _L), jnp.float32),
            jax.ShapeDtypeStruct((1, E), jnp.int32),
        ),
    )
    cs0, cs1, ds0, ds1, w0b, w1b, counts = router(xf, Wg)
    cs0f = cs0[:, 0]
    cs1f = cs1[:, 0]
    ds0f = ds0[:, 0]
    ds1f = ds1[:, 0]

    sub_d = 64
    nsub_d = (T // _NW) // sub_d
    mesh = plsc.VectorSubcoreMesh(
        core_axis_name="c", subcore_axis_name="s",
        num_cores=_NC, num_subcores=_NS)
    dispatch = functools.partial(
        pl.kernel,
        functools.partial(_dispatch_body, T, D, sub_d, nsub_d),
        out_type=jax.ShapeDtypeStruct((E * cap + _TILE, D), jnp.float32),
        mesh=mesh,
        scratch_types=[
            pltpu.VMEM((sub_d, D), jnp.float32),
            pltpu.VMEM((sub_d,), jnp.int32),
            pltpu.VMEM((sub_d,), jnp.int32),
            pltpu.SemaphoreType.DMA,
            pltpu.SemaphoreType.DMA,
        ],
    )()
    ein = dispatch(xf, ds0f, ds1f)

    expert = pl.pallas_call(
        functools.partial(_expert_body, cap, nt),
        grid=(E * nt,),
        in_specs=[
            pl.BlockSpec(memory_space=pltpu.SMEM),
            pl.BlockSpec((_TILE, D), lambda i: (i, 0)),
            pl.BlockSpec((1, D, H), lambda i: (i // nt, 0, 0)),
            pl.BlockSpec((1, 1, H), lambda i: (i // nt, 0, 0)),
            pl.BlockSpec((1, H, D), lambda i: (i // nt, 0, 0)),
            pl.BlockSpec((1, 1, D), lambda i: (i // nt, 0, 0)),
        ],
        out_specs=pl.BlockSpec((_TILE, D), lambda i: (i, 0)),
        out_shape=jax.ShapeDtypeStruct((E * cap, D), jnp.float32),
    )
    eout = expert(counts, ein, W1, b1.reshape(E, 1, H), W2,
                  b2.reshape(E, 1, D))

    sub_c = 32
    nsub_c = (T // _NW) // sub_c
    combine = functools.partial(
        pl.kernel,
        functools.partial(_combine_body, T, D, sub_c, nsub_c),
        out_type=jax.ShapeDtypeStruct((T, D), jnp.float32),
        mesh=mesh,
        scratch_types=[
            pltpu.VMEM((sub_c, D), jnp.float32),
            pltpu.VMEM((sub_c, D), jnp.float32),
            pltpu.VMEM((sub_c,), jnp.int32),
            pltpu.VMEM((sub_c,), jnp.int32),
            pltpu.VMEM((sub_c, _L), jnp.float32),
            pltpu.VMEM((sub_c, _L), jnp.float32),
            pltpu.SemaphoreType.DMA,
            pltpu.SemaphoreType.DMA,
        ],
    )()
    y = combine(eout, cs0f, cs1f, w0b, w1b)
    return y.reshape(Bx, Sx, D)


# speedup vs baseline: 1.0153x; 1.0153x over previous
import functools

import jax
import jax.numpy as jnp
import numpy as np
from jax import lax
from jax.experimental import pallas as pl
from jax.experimental.pallas import tpu as pltpu
from jax.experimental.pallas import tpu_sc as plsc

_K = 2
_CAPF = 1.25

_NC = 2
_NS = 16
_NW = _NC * _NS
_L = 16

_TILE = 256


def _router_body(cap, T, E, x_ref, wg_ref, cs0_ref, cs1_ref, ds0_ref, ds1_ref,
                 w0_ref, w1_ref, cnt_ref):
    logits = jnp.dot(x_ref[...], wg_ref[...], preferred_element_type=jnp.float32)
    iota_e = lax.broadcasted_iota(jnp.int32, (T, E), 1)
    m = jnp.max(logits, axis=1, keepdims=True)
    p = jnp.exp(logits - m)
    p1 = jnp.max(p, axis=1, keepdims=True)
    i1 = jnp.min(jnp.where(p == p1, iota_e, E), axis=1, keepdims=True)
    pm = jnp.where(iota_e == i1, -1.0, p)
    p2 = jnp.max(pm, axis=1, keepdims=True)
    i2 = jnp.min(jnp.where(pm == p2, iota_e, E), axis=1, keepdims=True)
    denom = p1 + p2
    w0 = p1 / denom
    w1 = p2 / denom

    oh2 = ((iota_e == i1) | (iota_e == i2)).astype(jnp.int32)
    c = oh2
    sh = 1
    while sh < T:
        c = c + jnp.concatenate(
            [jnp.zeros((sh, E), jnp.int32), c[:-sh]], axis=0)
        sh *= 2
    excl = c - oh2

    pos0 = jnp.sum(jnp.where(iota_e == i1, excl, 0), axis=1, keepdims=True)
    pos1 = jnp.sum(jnp.where(iota_e == i2, excl, 0), axis=1, keepdims=True)
    keep0 = pos0 < cap
    keep1 = pos1 < cap
    slot0 = i1 * cap + pos0
    slot1 = i2 * cap + pos1
    trash = E * cap

    cs0_ref[...] = jnp.broadcast_to(jnp.where(keep0, slot0, 0), (T, E))
    cs1_ref[...] = jnp.broadcast_to(jnp.where(keep1, slot1, 0), (T, E))
    ds0_ref[...] = jnp.broadcast_to(jnp.where(keep0, slot0, trash), (T, E))
    ds1_ref[...] = jnp.broadcast_to(jnp.where(keep1, slot1, trash), (T, E))
    w0_ref[...] = jnp.broadcast_to(jnp.where(keep0, w0, 0.0), (T, _L))
    w1_ref[...] = jnp.broadcast_to(jnp.where(keep1, w1, 0.0), (T, _L))
    counts = c[T - 1:T, :]
    cnt_ref[...] = jnp.minimum(counts, cap)


def _expert_body(cap, nt, cnt_ref, ein_ref, w1_ref, b1_ref, w2_ref, b2_ref,
                 out_ref):
    i = pl.program_id(0)
    e = i // nt
    tile_start = (i % nt) * _TILE
    nvalid = cnt_ref[0, e] - tile_start

    @pl.when(nvalid > 0)
    def _compute():
        xt = ein_ref[...]
        h = jnp.dot(xt, w1_ref[0], preferred_element_type=jnp.float32)
        h = jax.nn.gelu(h + b1_ref[0])
        o = jnp.dot(h, w2_ref[0], preferred_element_type=jnp.float32)
        o = o + b2_ref[0]
        rows = lax.broadcasted_iota(jnp.int32, (_TILE, 1), 0)
        out_ref[...] = jnp.where(rows < nvalid, o, 0.0)

    @pl.when(nvalid <= 0)
    def _zero():
        out_ref[...] = jnp.zeros_like(out_ref)


def _dispatch_body(T, D, sub, nsub, xf_hbm, d0_hbm, d1_hbm, ein_hbm,
                   xbuf, idx0, idx1, sem0, sem1):
    wid = lax.axis_index("s") * _NC + lax.axis_index("c")
    tpw = T // _NW

    def body(s, carry):
        base = wid * tpw + s * sub
        pltpu.sync_copy(d0_hbm.at[pl.ds(base, sub)], idx0)
        pltpu.sync_copy(d1_hbm.at[pl.ds(base, sub)], idx1)
        pltpu.sync_copy(xf_hbm.at[pl.ds(base, sub)], xbuf)
        cp0 = pltpu.async_copy(xbuf, ein_hbm.at[idx0], sem0)
        cp1 = pltpu.async_copy(xbuf, ein_hbm.at[idx1], sem1)
        cp0.wait()
        cp1.wait()
        return carry

    lax.fori_loop(0, nsub, body, 0)


def _combine_body(T, D, sub, nsub, eout_hbm, c0_hbm, c1_hbm, w0_hbm, w1_hbm,
                  y_hbm, r0, r1, idx0, idx1, w0v, w1v, sem0, sem1):
    wid = lax.axis_index("s") * _NC + lax.axis_index("c")
    tpw = T // _NW
    nd = D // _L

    def body(s, carry):
        base = wid * tpw + s * sub
        pltpu.sync_copy(c0_hbm.at[pl.ds(base, sub)], idx0)
        pltpu.sync_copy(c1_hbm.at[pl.ds(base, sub)], idx1)
        pltpu.sync_copy(w0_hbm.at[pl.ds(base, sub)], w0v)
        pltpu.sync_copy(w1_hbm.at[pl.ds(base, sub)], w1v)
        cp0 = pltpu.async_copy(eout_hbm.at[idx0], r0, sem0)
        cp1 = pltpu.async_copy(eout_hbm.at[idx1], r1, sem1)
        cp0.wait()
        cp1.wait()

        def tok_body(i, tc):
            a = w0v[i, pl.ds(0, _L)]
            b = w1v[i, pl.ds(0, _L)]

            def d_body(d, dc):
                for u in range(4):
                    off = d * (4 * _L) + u * _L
                    v = a * r0[i, pl.ds(off, _L)] + b * r1[i, pl.ds(off, _L)]
                    r0[i, pl.ds(off, _L)] = v
                return dc

            lax.fori_loop(0, nd // 4, d_body, 0)
            return tc

        lax.fori_loop(0, sub, tok_body, 0)
        pltpu.sync_copy(r0, y_hbm.at[pl.ds(base, sub)])
        return carry

    lax.fori_loop(0, nsub, body, 0)


def kernel(x, Wg, W1, b1, W2, b2):
    Bx, Sx, D = x.shape
    T = Bx * Sx
    E = Wg.shape[1]
    H = W1.shape[2]
    cap = int(np.ceil(T * _K / E * _CAPF))
    nt = cap // _TILE
    xf = x.reshape(T, D)

    router = pl.pallas_call(
        functools.partial(_router_body, cap, T, E),
        out_shape=(
            jax.ShapeDtypeStruct((T, E), jnp.int32),
            jax.ShapeDtypeStruct((T, E), jnp.int32),
            jax.ShapeDtypeStruct((T, E), jnp.int32),
            jax.ShapeDtypeStruct((T, E), jnp.int32),
            jax.ShapeDtypeStruct((T, _L), jnp.float32),
            jax.ShapeDtypeStruct((T, _L), jnp.float32),
            jax.ShapeDtypeStruct((1, E), jnp.int32),
        ),
    )
    cs0, cs1, ds0, ds1, w0b, w1b, counts = router(xf, Wg)
    cs0f = cs0[:, 0]
    cs1f = cs1[:, 0]
    ds0f = ds0[:, 0]
    ds1f = ds1[:, 0]

    sub_d = 64
    nsub_d = (T // _NW) // sub_d
    mesh = plsc.VectorSubcoreMesh(
        core_axis_name="c", subcore_axis_name="s",
        num_cores=_NC, num_subcores=_NS)
    dispatch = functools.partial(
        pl.kernel,
        functools.partial(_dispatch_body, T, D, sub_d, nsub_d),
        out_type=jax.ShapeDtypeStruct((E * cap + _TILE, D), jnp.float32),
        mesh=mesh,
        scratch_types=[
            pltpu.VMEM((sub_d, D), jnp.float32),
            pltpu.VMEM((sub_d,), jnp.int32),
            pltpu.VMEM((sub_d,), jnp.int32),
            pltpu.SemaphoreType.DMA,
            pltpu.SemaphoreType.DMA,
        ],
    )()
    ein = dispatch(xf, ds0f, ds1f)

    expert = pl.pallas_call(
        functools.partial(_expert_body, cap, nt),
        grid=(E * nt,),
        in_specs=[
            pl.BlockSpec(memory_space=pltpu.SMEM),
            pl.BlockSpec((_TILE, D), lambda i: (i, 0)),
            pl.BlockSpec((1, D, H), lambda i: (i // nt, 0, 0)),
            pl.BlockSpec((1, 1, H), lambda i: (i // nt, 0, 0)),
            pl.BlockSpec((1, H, D), lambda i: (i // nt, 0, 0)),
            pl.BlockSpec((1, 1, D), lambda i: (i // nt, 0, 0)),
        ],
        out_specs=pl.BlockSpec((_TILE, D), lambda i: (i, 0)),
        out_shape=jax.ShapeDtypeStruct((E * cap, D), jnp.float32),
    )
    eout = expert(counts, ein, W1, b1.reshape(E, 1, H), W2,
                  b2.reshape(E, 1, D))

    sub_c = 32
    nsub_c = (T // _NW) // sub_c
    combine = functools.partial(
        pl.kernel,
        functools.partial(_combine_body, T, D, sub_c, nsub_c),
        out_type=jax.ShapeDtypeStruct((T, D), jnp.float32),
        mesh=mesh,
        scratch_types=[
            pltpu.VMEM((sub_c, D), jnp.float32),
            pltpu.VMEM((sub_c, D), jnp.float32),
            pltpu.VMEM((sub_c,), jnp.int32),
            pltpu.VMEM((sub_c,), jnp.int32),
            pltpu.VMEM((sub_c, _L), jnp.float32),
            pltpu.VMEM((sub_c, _L), jnp.float32),
            pltpu.SemaphoreType.DMA,
            pltpu.SemaphoreType.DMA,
        ],
    )()
    y = combine(eout, cs0f, cs1f, w0b, w1b)
    return y.reshape(Bx, Sx, D)


# speedup vs baseline: 1.1135x; 1.0967x over previous
import functools

import jax
import jax.numpy as jnp
import numpy as np
from jax import lax
from jax.experimental import pallas as pl
from jax.experimental.pallas import tpu as pltpu
from jax.experimental.pallas import tpu_sc as plsc

_K = 2
_CAPF = 1.25

_NC = 2
_NS = 16
_NW = _NC * _NS
_L = 16

_TILE = 256


def _router_body(cap, T, E, x_ref, wg_ref, cs0_ref, cs1_ref, ds0_ref, ds1_ref,
                 w0_ref, w1_ref, cnt_ref):
    logits = jnp.dot(x_ref[...], wg_ref[...], preferred_element_type=jnp.float32)
    iota_e = lax.broadcasted_iota(jnp.int32, (T, E), 1)
    m = jnp.max(logits, axis=1, keepdims=True)
    p = jnp.exp(logits - m)
    p1 = jnp.max(p, axis=1, keepdims=True)
    i1 = jnp.min(jnp.where(p == p1, iota_e, E), axis=1, keepdims=True)
    pm = jnp.where(iota_e == i1, -1.0, p)
    p2 = jnp.max(pm, axis=1, keepdims=True)
    i2 = jnp.min(jnp.where(pm == p2, iota_e, E), axis=1, keepdims=True)
    denom = p1 + p2
    w0 = p1 / denom
    w1 = p2 / denom

    oh2 = ((iota_e == i1) | (iota_e == i2)).astype(jnp.int32)
    c = oh2
    sh = 1
    while sh < T:
        c = c + jnp.concatenate(
            [jnp.zeros((sh, E), jnp.int32), c[:-sh]], axis=0)
        sh *= 2
    excl = c - oh2

    pos0 = jnp.sum(jnp.where(iota_e == i1, excl, 0), axis=1, keepdims=True)
    pos1 = jnp.sum(jnp.where(iota_e == i2, excl, 0), axis=1, keepdims=True)
    keep0 = pos0 < cap
    keep1 = pos1 < cap
    slot0 = i1 * cap + pos0
    slot1 = i2 * cap + pos1
    trash = E * cap

    cs0_ref[...] = jnp.broadcast_to(jnp.where(keep0, slot0, 0), (T, E))
    cs1_ref[...] = jnp.broadcast_to(jnp.where(keep1, slot1, 0), (T, E))
    ds0_ref[...] = jnp.broadcast_to(jnp.where(keep0, slot0, trash), (T, E))
    ds1_ref[...] = jnp.broadcast_to(jnp.where(keep1, slot1, trash), (T, E))
    w0_ref[...] = jnp.broadcast_to(jnp.where(keep0, w0, 0.0), (T, _L))
    w1_ref[...] = jnp.broadcast_to(jnp.where(keep1, w1, 0.0), (T, _L))
    counts = c[T - 1:T, :]
    cnt_ref[...] = jnp.minimum(counts, cap)


def _expert_body(cap, nt, cnt_ref, ein_ref, w1_ref, b1_ref, w2_ref, b2_ref,
                 out_ref):
    i = pl.program_id(0)
    e = i // nt
    tile_start = (i % nt) * _TILE
    nvalid = cnt_ref[0, e] - tile_start

    @pl.when(nvalid > 0)
    def _compute():
        xt = ein_ref[...]
        h = jnp.dot(xt, w1_ref[0], preferred_element_type=jnp.float32)
        h = jax.nn.gelu(h + b1_ref[0])
        o = jnp.dot(h, w2_ref[0], preferred_element_type=jnp.float32)
        o = o + b2_ref[0]
        rows = lax.broadcasted_iota(jnp.int32, (_TILE, 1), 0)
        out_ref[...] = jnp.where(rows < nvalid, o, 0.0)

    @pl.when(nvalid <= 0)
    def _zero():
        out_ref[...] = jnp.zeros_like(out_ref)


def _dispatch_body(T, D, sub, nsub, xf_hbm, d0_hbm, d1_hbm, ein_hbm,
                   xbuf, idx0, idx1, sem0, sem1):
    wid = lax.axis_index("s") * _NC + lax.axis_index("c")
    tpw = T // _NW

    def body(s, carry):
        base = wid * tpw + s * sub
        pltpu.sync_copy(d0_hbm.at[pl.ds(base, sub)], idx0)
        pltpu.sync_copy(d1_hbm.at[pl.ds(base, sub)], idx1)
        pltpu.sync_copy(xf_hbm.at[pl.ds(base, sub)], xbuf)
        cp0 = pltpu.async_copy(xbuf, ein_hbm.at[idx0], sem0)
        cp1 = pltpu.async_copy(xbuf, ein_hbm.at[idx1], sem1)
        cp0.wait()
        cp1.wait()
        return carry

    lax.fori_loop(0, nsub, body, 0)


def _combine_body(T, D, sub, nsub, eout_hbm, c0_hbm, c1_hbm, w0_hbm, w1_hbm,
                  y_hbm, r0, r1, yb, idx0, idx1, w0v, w1v, sem0, sem1):
    wid = lax.axis_index("s") * _NC + lax.axis_index("c")
    tpw = T // _NW
    nd = D // _L

    def body(s, carry):
        base = wid * tpw + s * sub
        pltpu.sync_copy(c0_hbm.at[pl.ds(base, sub)], idx0)
        pltpu.sync_copy(c1_hbm.at[pl.ds(base, sub)], idx1)
        pltpu.sync_copy(w0_hbm.at[pl.ds(base, sub)], w0v)
        pltpu.sync_copy(w1_hbm.at[pl.ds(base, sub)], w1v)
        cp0 = pltpu.async_copy(eout_hbm.at[idx0], r0, sem0)
        cp1 = pltpu.async_copy(eout_hbm.at[idx1], r1, sem1)
        cp0.wait()
        cp1.wait()

        def tok_body(i, tc):
            a = w0v[i, pl.ds(0, _L)]
            b = w1v[i, pl.ds(0, _L)]

            def d_body(d, dc):
                off = d * _L
                v = a * r0[i, pl.ds(off, _L)] + b * r1[i, pl.ds(off, _L)]
                yb[i, pl.ds(off, _L)] = v
                return dc

            lax.fori_loop(0, nd, d_body, 0)
            return tc

        lax.fori_loop(0, sub, tok_body, 0)
        pltpu.sync_copy(yb, y_hbm.at[pl.ds(base, sub)])
        return carry

    lax.fori_loop(0, nsub, body, 0)


def kernel(x, Wg, W1, b1, W2, b2):
    Bx, Sx, D = x.shape
    T = Bx * Sx
    E = Wg.shape[1]
    H = W1.shape[2]
    cap = int(np.ceil(T * _K / E * _CAPF))
    nt = cap // _TILE
    xf = x.reshape(T, D)

    router = pl.pallas_call(
        functools.partial(_router_body, cap, T, E),
        out_shape=(
            jax.ShapeDtypeStruct((T, E), jnp.int32),
            jax.ShapeDtypeStruct((T, E), jnp.int32),
            jax.ShapeDtypeStruct((T, E), jnp.int32),
            jax.ShapeDtypeStruct((T, E), jnp.int32),
            jax.ShapeDtypeStruct((T, _L), jnp.float32),
            jax.ShapeDtypeStruct((T, _L), jnp.float32),
            jax.ShapeDtypeStruct((1, E), jnp.int32),
        ),
    )
    cs0, cs1, ds0, ds1, w0b, w1b, counts = router(xf, Wg)
    cs0f = cs0[:, 0]
    cs1f = cs1[:, 0]
    ds0f = ds0[:, 0]
    ds1f = ds1[:, 0]

    sub_d = 64
    nsub_d = (T // _NW) // sub_d
    mesh = plsc.VectorSubcoreMesh(
        core_axis_name="c", subcore_axis_name="s",
        num_cores=_NC, num_subcores=_NS)
    dispatch = functools.partial(
        pl.kernel,
        functools.partial(_dispatch_body, T, D, sub_d, nsub_d),
        out_type=jax.ShapeDtypeStruct((E * cap + _TILE, D), jnp.float32),
        mesh=mesh,
        scratch_types=[
            pltpu.VMEM((sub_d, D), jnp.float32),
            pltpu.VMEM((sub_d,), jnp.int32),
            pltpu.VMEM((sub_d,), jnp.int32),
            pltpu.SemaphoreType.DMA,
            pltpu.SemaphoreType.DMA,
        ],
    )()
    ein = dispatch(xf, ds0f, ds1f)

    expert = pl.pallas_call(
        functools.partial(_expert_body, cap, nt),
        grid=(E * nt,),
        in_specs=[
            pl.BlockSpec(memory_space=pltpu.SMEM),
            pl.BlockSpec((_TILE, D), lambda i: (i, 0)),
            pl.BlockSpec((1, D, H), lambda i: (i // nt, 0, 0)),
            pl.BlockSpec((1, 1, H), lambda i: (i // nt, 0, 0)),
            pl.BlockSpec((1, H, D), lambda i: (i // nt, 0, 0)),
            pl.BlockSpec((1, 1, D), lambda i: (i // nt, 0, 0)),
        ],
        out_specs=pl.BlockSpec((_TILE, D), lambda i: (i, 0)),
        out_shape=jax.ShapeDtypeStruct((E * cap, D), jnp.float32),
    )
    eout = expert(counts, ein, W1, b1.reshape(E, 1, H), W2,
                  b2.reshape(E, 1, D))

    sub_c = 32
    nsub_c = (T // _NW) // sub_c
    combine = functools.partial(
        pl.kernel,
        functools.partial(_combine_body, T, D, sub_c, nsub_c),
        out_type=jax.ShapeDtypeStruct((T, D), jnp.float32),
        mesh=mesh,
        scratch_types=[
            pltpu.VMEM((sub_c, D), jnp.float32),
            pltpu.VMEM((sub_c, D), jnp.float32),
            pltpu.VMEM((sub_c, D), jnp.float32),
            pltpu.VMEM((sub_c,), jnp.int32),
            pltpu.VMEM((sub_c,), jnp.int32),
            pltpu.VMEM((sub_c, _L), jnp.float32),
            pltpu.VMEM((sub_c, _L), jnp.float32),
            pltpu.SemaphoreType.DMA,
            pltpu.SemaphoreType.DMA,
        ],
    )()
    y = combine(eout, cs0f, cs1f, w0b, w1b)
    return y.reshape(Bx, Sx, D)


# speedup vs baseline: 1.1186x; 1.0046x over previous
import functools

import jax
import jax.numpy as jnp
import numpy as np
from jax import lax
from jax.experimental import pallas as pl
from jax.experimental.pallas import tpu as pltpu
from jax.experimental.pallas import tpu_sc as plsc

_K = 2
_CAPF = 1.25

_NC = 2
_NS = 16
_NW = _NC * _NS
_L = 16

_TILE = 256
_WROW = 128


def _router_body(cap, T, E, x_ref, wg_ref, cs0_ref, cs1_ref, ds0_ref, ds1_ref,
                 w0_ref, w1_ref, cnt_ref):
    logits = jnp.dot(x_ref[...], wg_ref[...], preferred_element_type=jnp.float32)
    iota_e = lax.broadcasted_iota(jnp.int32, (T, E), 1)
    m = jnp.max(logits, axis=1, keepdims=True)
    p = jnp.exp(logits - m)
    p1 = jnp.max(p, axis=1, keepdims=True)
    i1 = jnp.min(jnp.where(p == p1, iota_e, E), axis=1, keepdims=True)
    pm = jnp.where(iota_e == i1, -1.0, p)
    p2 = jnp.max(pm, axis=1, keepdims=True)
    i2 = jnp.min(jnp.where(pm == p2, iota_e, E), axis=1, keepdims=True)
    denom = p1 + p2
    w0 = p1 / denom
    w1 = p2 / denom

    oh2 = ((iota_e == i1) | (iota_e == i2)).astype(jnp.int32)
    c = oh2
    sh = 1
    while sh < T:
        c = c + jnp.concatenate(
            [jnp.zeros((sh, E), jnp.int32), c[:-sh]], axis=0)
        sh *= 2
    excl = c - oh2

    pos0 = jnp.sum(jnp.where(iota_e == i1, excl, 0), axis=1, keepdims=True)
    pos1 = jnp.sum(jnp.where(iota_e == i2, excl, 0), axis=1, keepdims=True)
    keep0 = pos0 < cap
    keep1 = pos1 < cap
    slot0 = i1 * cap + pos0
    slot1 = i2 * cap + pos1
    trash = E * cap

    cs0_ref[...] = jnp.broadcast_to(jnp.where(keep0, slot0, trash), (T, E))
    cs1_ref[...] = jnp.broadcast_to(jnp.where(keep1, slot1, trash), (T, E))
    ds0_ref[...] = jnp.broadcast_to(jnp.where(keep0, slot0, trash), (T, E))
    ds1_ref[...] = jnp.broadcast_to(jnp.where(keep1, slot1, trash), (T, E))
    w0_ref[...] = jnp.broadcast_to(jnp.where(keep0, w0, 0.0), (T, _WROW))
    w1_ref[...] = jnp.broadcast_to(jnp.where(keep1, w1, 0.0), (T, _WROW))
    counts = c[T - 1:T, :]
    cnt_ref[...] = jnp.minimum(counts, cap)


def _expert_body(cap, nt, E, cnt_ref, ein_ref, ws_ref, w1_ref, b1_ref, w2_ref,
                 b2_ref, out_ref):
    i = pl.program_id(0)
    e = jnp.minimum(i // nt, E - 1)
    tile_start = (i % nt) * _TILE
    nvalid = jnp.where(i >= E * nt, -1, cnt_ref[0, e] - tile_start)

    @pl.when(nvalid > 0)
    def _compute():
        xt = ein_ref[...]
        h = jnp.dot(xt, w1_ref[0], preferred_element_type=jnp.float32)
        h = jax.nn.gelu(h + b1_ref[0])
        o = jnp.dot(h, w2_ref[0], preferred_element_type=jnp.float32)
        o = (o + b2_ref[0]) * ws_ref[:, 0:1]
        rows = lax.broadcasted_iota(jnp.int32, (_TILE, 1), 0)
        out_ref[...] = jnp.where(rows < nvalid, o, 0.0)

    @pl.when(nvalid <= 0)
    def _zero():
        out_ref[...] = jnp.zeros_like(out_ref)


def _dispatch_body(T, D, sub, nsub, xf_hbm, d0_hbm, d1_hbm, w0_hbm, w1_hbm,
                   ein_hbm, ws_hbm, xbuf, idx0, idx1, wb0, wb1, sem0, sem1,
                   sem2, sem3):
    wid = lax.axis_index("s") * _NC + lax.axis_index("c")
    tpw = T // _NW

    def body(s, carry):
        base = wid * tpw + s * sub
        pltpu.sync_copy(d0_hbm.at[pl.ds(base, sub)], idx0)
        pltpu.sync_copy(d1_hbm.at[pl.ds(base, sub)], idx1)
        pltpu.sync_copy(w0_hbm.at[pl.ds(base, sub)], wb0)
        pltpu.sync_copy(w1_hbm.at[pl.ds(base, sub)], wb1)
        pltpu.sync_copy(xf_hbm.at[pl.ds(base, sub)], xbuf)
        cp0 = pltpu.async_copy(xbuf, ein_hbm.at[idx0], sem0)
        cp1 = pltpu.async_copy(xbuf, ein_hbm.at[idx1], sem1)
        cp2 = pltpu.async_copy(wb0, ws_hbm.at[idx0], sem2)
        cp3 = pltpu.async_copy(wb1, ws_hbm.at[idx1], sem3)
        cp0.wait()
        cp1.wait()
        cp2.wait()
        cp3.wait()
        return carry

    lax.fori_loop(0, nsub, body, 0)


def _combine_body(T, D, sub, nsub, eout_hbm, c0_hbm, c1_hbm, y_hbm,
                  r0, r1, yb, idx0, idx1, sem0, sem1):
    wid = lax.axis_index("s") * _NC + lax.axis_index("c")
    tpw = T // _NW
    nd = D // _L

    def body(s, carry):
        base = wid * tpw + s * sub
        pltpu.sync_copy(c0_hbm.at[pl.ds(base, sub)], idx0)
        pltpu.sync_copy(c1_hbm.at[pl.ds(base, sub)], idx1)
        cp0 = pltpu.async_copy(eout_hbm.at[idx0], r0, sem0)
        cp1 = pltpu.async_copy(eout_hbm.at[idx1], r1, sem1)
        cp0.wait()
        cp1.wait()

        def tok_body(i, tc):
            def d_body(d, dc):
                off = d * _L
                yb[i, pl.ds(off, _L)] = r0[i, pl.ds(off, _L)] + r1[i, pl.ds(off, _L)]
                return dc

            lax.fori_loop(0, nd, d_body, 0)
            return tc

        lax.fori_loop(0, sub, tok_body, 0)
        pltpu.sync_copy(yb, y_hbm.at[pl.ds(base, sub)])
        return carry

    lax.fori_loop(0, nsub, body, 0)


def kernel(x, Wg, W1, b1, W2, b2):
    Bx, Sx, D = x.shape
    T = Bx * Sx
    E = Wg.shape[1]
    H = W1.shape[2]
    cap = int(np.ceil(T * _K / E * _CAPF))
    nt = cap // _TILE
    xf = x.reshape(T, D)

    router = pl.pallas_call(
        functools.partial(_router_body, cap, T, E),
        out_shape=(
            jax.ShapeDtypeStruct((T, E), jnp.int32),
            jax.ShapeDtypeStruct((T, E), jnp.int32),
            jax.ShapeDtypeStruct((T, E), jnp.int32),
            jax.ShapeDtypeStruct((T, E), jnp.int32),
            jax.ShapeDtypeStruct((T, _WROW), jnp.float32),
            jax.ShapeDtypeStruct((T, _WROW), jnp.float32),
            jax.ShapeDtypeStruct((1, E), jnp.int32),
        ),
    )
    cs0, cs1, ds0, ds1, w0b, w1b, counts = router(xf, Wg)
    cs0f = cs0[:, 0]
    cs1f = cs1[:, 0]
    ds0f = ds0[:, 0]
    ds1f = ds1[:, 0]

    sub_d = 64
    nsub_d = (T // _NW) // sub_d
    mesh = plsc.VectorSubcoreMesh(
        core_axis_name="c", subcore_axis_name="s",
        num_cores=_NC, num_subcores=_NS)
    dispatch = functools.partial(
        pl.kernel,
        functools.partial(_dispatch_body, T, D, sub_d, nsub_d),
        out_type=(
            jax.ShapeDtypeStruct((E * cap + _TILE, D), jnp.float32),
            jax.ShapeDtypeStruct((E * cap + _TILE, _WROW), jnp.float32),
        ),
        mesh=mesh,
        scratch_types=[
            pltpu.VMEM((sub_d, D), jnp.float32),
            pltpu.VMEM((sub_d,), jnp.int32),
            pltpu.VMEM((sub_d,), jnp.int32),
            pltpu.VMEM((sub_d, _WROW), jnp.float32),
            pltpu.VMEM((sub_d, _WROW), jnp.float32),
            pltpu.SemaphoreType.DMA,
            pltpu.SemaphoreType.DMA,
            pltpu.SemaphoreType.DMA,
            pltpu.SemaphoreType.DMA,
        ],
    )()
    ein, wslot = dispatch(xf, ds0f, ds1f, w0b, w1b)

    ew = lambda i: (jnp.minimum(i // nt, E - 1), 0, 0)
    expert = pl.pallas_call(
        functools.partial(_expert_body, cap, nt, E),
        grid=(E * nt + 1,),
        in_specs=[
            pl.BlockSpec(memory_space=pltpu.SMEM),
            pl.BlockSpec((_TILE, D), lambda i: (i, 0)),
            pl.BlockSpec((_TILE, _WROW), lambda i: (i, 0)),
            pl.BlockSpec((1, D, H), ew),
            pl.BlockSpec((1, 1, H), ew),
            pl.BlockSpec((1, H, D), ew),
            pl.BlockSpec((1, 1, D), ew),
        ],
        out_specs=pl.BlockSpec((_TILE, D), lambda i: (i, 0)),
        out_shape=jax.ShapeDtypeStruct((E * cap + _TILE, D), jnp.float32),
    )
    eout = expert(counts, ein, wslot, W1, b1.reshape(E, 1, H), W2,
                  b2.reshape(E, 1, D))

    sub_c = 32
    nsub_c = (T // _NW) // sub_c
    combine = functools.partial(
        pl.kernel,
        functools.partial(_combine_body, T, D, sub_c, nsub_c),
        out_type=jax.ShapeDtypeStruct((T, D), jnp.float32),
        mesh=mesh,
        scratch_types=[
            pltpu.VMEM((sub_c, D), jnp.float32),
            pltpu.VMEM((sub_c, D), jnp.float32),
            pltpu.VMEM((sub_c, D), jnp.float32),
            pltpu.VMEM((sub_c,), jnp.int32),
            pltpu.VMEM((sub_c,), jnp.int32),
            pltpu.SemaphoreType.DMA,
            pltpu.SemaphoreType.DMA,
        ],
    )()
    y = combine(eout, cs0f, cs1f)
    return y.reshape(Bx, Sx, D)


# speedup vs baseline: 1.1873x; 1.0615x over previous
import functools

import jax
import jax.numpy as jnp
import numpy as np
from jax import lax
from jax.experimental import pallas as pl
from jax.experimental.pallas import tpu as pltpu
from jax.experimental.pallas import tpu_sc as plsc

_K = 2
_CAPF = 1.25

_NC = 2
_NS = 16
_NW = _NC * _NS
_L = 16

_TILE = 256
_WROW = 128


def _router_body(cap, T, E, x_ref, wg_ref, ds0_ref, ds1_ref,
                 w0_ref, w1_ref, cnt_ref):
    logits = jnp.dot(x_ref[...], wg_ref[...], preferred_element_type=jnp.float32)
    iota_e = lax.broadcasted_iota(jnp.int32, (T, E), 1)
    m = jnp.max(logits, axis=1, keepdims=True)
    p = jnp.exp(logits - m)
    p1 = jnp.max(p, axis=1, keepdims=True)
    i1 = jnp.min(jnp.where(p == p1, iota_e, E), axis=1, keepdims=True)
    pm = jnp.where(iota_e == i1, -1.0, p)
    p2 = jnp.max(pm, axis=1, keepdims=True)
    i2 = jnp.min(jnp.where(pm == p2, iota_e, E), axis=1, keepdims=True)
    denom = p1 + p2
    w0 = p1 / denom
    w1 = p2 / denom

    oh2 = ((iota_e == i1) | (iota_e == i2)).astype(jnp.int32)
    c = oh2
    sh = 1
    while sh < T:
        c = c + jnp.concatenate(
            [jnp.zeros((sh, E), jnp.int32), c[:-sh]], axis=0)
        sh *= 2
    excl = c - oh2

    pos0 = jnp.sum(jnp.where(iota_e == i1, excl, 0), axis=1, keepdims=True)
    pos1 = jnp.sum(jnp.where(iota_e == i2, excl, 0), axis=1, keepdims=True)
    keep0 = pos0 < cap
    keep1 = pos1 < cap
    slot0 = i1 * cap + pos0
    slot1 = i2 * cap + pos1
    trash = E * cap

    ds0_ref[...] = jnp.broadcast_to(jnp.where(keep0, slot0, trash), (T, E))
    ds1_ref[...] = jnp.broadcast_to(jnp.where(keep1, slot1, trash), (T, E))
    w0_ref[...] = jnp.broadcast_to(jnp.where(keep0, w0, 0.0), (T, _WROW))
    w1_ref[...] = jnp.broadcast_to(jnp.where(keep1, w1, 0.0), (T, _WROW))
    counts = c[T - 1:T, :]
    cnt_ref[...] = jnp.minimum(counts, cap)


def _expert_body(cap, nt, E, cnt_ref, ein_ref, ws_ref, w1_ref, b1_ref, w2_ref,
                 b2_ref, out_ref):
    i = pl.program_id(0)
    e = jnp.minimum(i // nt, E - 1)
    tile_start = (i % nt) * _TILE
    nvalid = jnp.where(i >= E * nt, -1, cnt_ref[0, e] - tile_start)

    @pl.when(nvalid > 0)
    def _compute():
        xt = ein_ref[...]
        h = jnp.dot(xt, w1_ref[0], preferred_element_type=jnp.float32)
        h = jax.nn.gelu(h + b1_ref[0])
        o = jnp.dot(h, w2_ref[0], preferred_element_type=jnp.float32)
        o = (o + b2_ref[0]) * ws_ref[:, 0:1]
        rows = lax.broadcasted_iota(jnp.int32, (_TILE, 1), 0)
        out_ref[...] = jnp.where(rows < nvalid, o, 0.0)

    @pl.when(nvalid <= 0)
    def _zero():
        out_ref[...] = jnp.zeros_like(out_ref)


def _dispatch_body(T, D, sub, nsub, xf_hbm, d0_hbm, d1_hbm, w0_hbm, w1_hbm,
                   ein_hbm, ws_hbm, xbuf, idx0, idx1, wb0, wb1, sem0, sem1,
                   sem2, sem3):
    wid = lax.axis_index("s") * _NC + lax.axis_index("c")
    tpw = T // _NW

    def body(s, carry):
        base = wid * tpw + s * sub
        pltpu.sync_copy(d0_hbm.at[pl.ds(base, sub)], idx0)
        pltpu.sync_copy(d1_hbm.at[pl.ds(base, sub)], idx1)
        pltpu.sync_copy(w0_hbm.at[pl.ds(base, sub)], wb0)
        pltpu.sync_copy(w1_hbm.at[pl.ds(base, sub)], wb1)
        pltpu.sync_copy(xf_hbm.at[pl.ds(base, sub)], xbuf)
        cp0 = pltpu.async_copy(xbuf, ein_hbm.at[idx0], sem0)
        cp1 = pltpu.async_copy(xbuf, ein_hbm.at[idx1], sem1)
        cp2 = pltpu.async_copy(wb0, ws_hbm.at[idx0], sem2)
        cp3 = pltpu.async_copy(wb1, ws_hbm.at[idx1], sem3)
        cp0.wait()
        cp1.wait()
        cp2.wait()
        cp3.wait()
        return carry

    lax.fori_loop(0, nsub, body, 0)


def _combine_body(T, D, sub, nsub, eout_hbm, c0_hbm, c1_hbm, y_hbm,
                  r0, r1, yb, idx0, idx1, sem0, sem1):
    wid = lax.axis_index("s") * _NC + lax.axis_index("c")
    tpw = T // _NW
    nd = D // _L

    def body(s, carry):
        base = wid * tpw + s * sub
        pltpu.sync_copy(c0_hbm.at[pl.ds(base, sub)], idx0)
        pltpu.sync_copy(c1_hbm.at[pl.ds(base, sub)], idx1)
        cp0 = pltpu.async_copy(eout_hbm.at[idx0], r0, sem0)
        cp1 = pltpu.async_copy(eout_hbm.at[idx1], r1, sem1)
        cp0.wait()
        cp1.wait()

        def tok_body(i, tc):
            for off in range(0, D, _L):
                yb[i, pl.ds(off, _L)] = (
                    r0[i, pl.ds(off, _L)] + r1[i, pl.ds(off, _L)])
            return tc

        lax.fori_loop(0, sub, tok_body, 0)
        pltpu.sync_copy(yb, y_hbm.at[pl.ds(base, sub)])
        return carry

    lax.fori_loop(0, nsub, body, 0)


def kernel(x, Wg, W1, b1, W2, b2):
    Bx, Sx, D = x.shape
    T = Bx * Sx
    E = Wg.shape[1]
    H = W1.shape[2]
    cap = int(np.ceil(T * _K / E * _CAPF))
    nt = cap // _TILE
    xf = x.reshape(T, D)

    router = pl.pallas_call(
        functools.partial(_router_body, cap, T, E),
        out_shape=(
            jax.ShapeDtypeStruct((T, E), jnp.int32),
            jax.ShapeDtypeStruct((T, E), jnp.int32),
            jax.ShapeDtypeStruct((T, _WROW), jnp.float32),
            jax.ShapeDtypeStruct((T, _WROW), jnp.float32),
            jax.ShapeDtypeStruct((1, E), jnp.int32),
        ),
    )
    ds0, ds1, w0b, w1b, counts = router(xf, Wg)
    ds0f = ds0[:, 0]
    ds1f = ds1[:, 0]

    sub_d = 64
    nsub_d = (T // _NW) // sub_d
    mesh = plsc.VectorSubcoreMesh(
        core_axis_name="c", subcore_axis_name="s",
        num_cores=_NC, num_subcores=_NS)
    dispatch = functools.partial(
        pl.kernel,
        functools.partial(_dispatch_body, T, D, sub_d, nsub_d),
        out_type=(
            jax.ShapeDtypeStruct((E * cap + _TILE, D), jnp.float32),
            jax.ShapeDtypeStruct((E * cap + _TILE, _WROW), jnp.float32),
        ),
        mesh=mesh,
        scratch_types=[
            pltpu.VMEM((sub_d, D), jnp.float32),
            pltpu.VMEM((sub_d,), jnp.int32),
            pltpu.VMEM((sub_d,), jnp.int32),
            pltpu.VMEM((sub_d, _WROW), jnp.float32),
            pltpu.VMEM((sub_d, _WROW), jnp.float32),
            pltpu.SemaphoreType.DMA,
            pltpu.SemaphoreType.DMA,
            pltpu.SemaphoreType.DMA,
            pltpu.SemaphoreType.DMA,
        ],
    )()
    ein, wslot = dispatch(xf, ds0f, ds1f, w0b, w1b)

    ew = lambda i: (jnp.minimum(i // nt, E - 1), 0, 0)
    expert = pl.pallas_call(
        functools.partial(_expert_body, cap, nt, E),
        grid=(E * nt + 1,),
        in_specs=[
            pl.BlockSpec(memory_space=pltpu.SMEM),
            pl.BlockSpec((_TILE, D), lambda i: (i, 0)),
            pl.BlockSpec((_TILE, _WROW), lambda i: (i, 0)),
            pl.BlockSpec((1, D, H), ew),
            pl.BlockSpec((1, 1, H), ew),
            pl.BlockSpec((1, H, D), ew),
            pl.BlockSpec((1, 1, D), ew),
        ],
        out_specs=pl.BlockSpec((_TILE, D), lambda i: (i, 0)),
        out_shape=jax.ShapeDtypeStruct((E * cap + _TILE, D), jnp.float32),
    )
    eout = expert(counts, ein, wslot, W1, b1.reshape(E, 1, H), W2,
                  b2.reshape(E, 1, D))

    sub_c = 32
    nsub_c = (T // _NW) // sub_c
    combine = functools.partial(
        pl.kernel,
        functools.partial(_combine_body, T, D, sub_c, nsub_c),
        out_type=jax.ShapeDtypeStruct((T, D), jnp.float32),
        mesh=mesh,
        scratch_types=[
            pltpu.VMEM((sub_c, D), jnp.float32),
            pltpu.VMEM((sub_c, D), jnp.float32),
            pltpu.VMEM((sub_c, D), jnp.float32),
            pltpu.VMEM((sub_c,), jnp.int32),
            pltpu.VMEM((sub_c,), jnp.int32),
            pltpu.SemaphoreType.DMA,
            pltpu.SemaphoreType.DMA,
        ],
    )()
    y = combine(eout, ds0f, ds1f)
    return y.reshape(Bx, Sx, D)


# speedup vs baseline: 1.2543x; 1.0564x over previous
import functools

import jax
import jax.numpy as jnp
import numpy as np
from jax import lax
from jax.experimental import pallas as pl
from jax.experimental.pallas import tpu as pltpu
from jax.experimental.pallas import tpu_sc as plsc

_K = 2
_CAPF = 1.25

_NC = 2
_NS = 16
_NW = _NC * _NS
_L = 16

_TILE = 256
_WROW = 128


def _router_body(cap, T, E, x_ref, wg_ref, ds0_ref, ds1_ref,
                 w0_ref, w1_ref, cnt_ref):
    logits = jnp.dot(x_ref[...], wg_ref[...], preferred_element_type=jnp.float32)
    iota_e = lax.broadcasted_iota(jnp.int32, (T, E), 1)
    m = jnp.max(logits, axis=1, keepdims=True)
    p = jnp.exp(logits - m)
    p1 = jnp.max(p, axis=1, keepdims=True)
    i1 = jnp.min(jnp.where(p == p1, iota_e, E), axis=1, keepdims=True)
    pm = jnp.where(iota_e == i1, -1.0, p)
    p2 = jnp.max(pm, axis=1, keepdims=True)
    i2 = jnp.min(jnp.where(pm == p2, iota_e, E), axis=1, keepdims=True)
    denom = p1 + p2
    w0 = p1 / denom
    w1 = p2 / denom

    oh2 = ((iota_e == i1) | (iota_e == i2)).astype(jnp.int32)
    c = oh2
    sh = 1
    while sh < T:
        c = c + jnp.concatenate(
            [jnp.zeros((sh, E), jnp.int32), c[:-sh]], axis=0)
        sh *= 2
    excl = c - oh2

    pos0 = jnp.sum(jnp.where(iota_e == i1, excl, 0), axis=1, keepdims=True)
    pos1 = jnp.sum(jnp.where(iota_e == i2, excl, 0), axis=1, keepdims=True)
    keep0 = pos0 < cap
    keep1 = pos1 < cap
    slot0 = i1 * cap + pos0
    slot1 = i2 * cap + pos1
    trash = E * cap

    ds0_ref[...] = jnp.broadcast_to(jnp.where(keep0, slot0, trash), (T, E))
    ds1_ref[...] = jnp.broadcast_to(jnp.where(keep1, slot1, trash), (T, E))
    w0_ref[...] = jnp.broadcast_to(jnp.where(keep0, w0, 0.0), (T, _WROW))
    w1_ref[...] = jnp.broadcast_to(jnp.where(keep1, w1, 0.0), (T, _WROW))
    counts = c[T - 1:T, :]
    cnt_ref[...] = jnp.minimum(counts, cap)


def _expert_body(cap, nt, E, cnt_ref, ein_ref, ws_ref, w1_ref, b1_ref, w2_ref,
                 b2_ref, out_ref):
    i = pl.program_id(0)
    e = jnp.minimum(i // nt, E - 1)
    tile_start = (i % nt) * _TILE
    nvalid = jnp.where(i >= E * nt, -1, cnt_ref[0, e] - tile_start)

    @pl.when(nvalid > 0)
    def _compute():
        xt = ein_ref[...]
        h = jnp.dot(xt, w1_ref[0], preferred_element_type=jnp.float32)
        h = jax.nn.gelu(h + b1_ref[0])
        o = jnp.dot(h, w2_ref[0], preferred_element_type=jnp.float32)
        o = (o + b2_ref[0]) * ws_ref[:, 0:1]
        rows = lax.broadcasted_iota(jnp.int32, (_TILE, 1), 0)
        out_ref[...] = jnp.where(rows < nvalid, o, 0.0)

    @pl.when(nvalid <= 0)
    def _zero():
        out_ref[...] = jnp.zeros_like(out_ref)


def _dispatch_body(T, D, sub, nsub, xf_hbm, d0_hbm, d1_hbm, w0_hbm, w1_hbm,
                   ein_hbm, ws_hbm, xbufa, xbufb, idx0, idx1, wb0, wb1,
                   insema, insemb, scsema, scsemb):
    wid = lax.axis_index("s") * _NC + lax.axis_index("c")
    tpw = T // _NW
    base0 = wid * tpw
    xbufs = (xbufa, xbufb)
    insems = (insema, insemb)
    scsems = (scsema, scsemb)

    pltpu.sync_copy(d0_hbm.at[pl.ds(wid * nsub, nsub)], idx0)
    pltpu.sync_copy(d1_hbm.at[pl.ds(wid * nsub, nsub)], idx1)
    pltpu.sync_copy(w0_hbm.at[pl.ds(base0, tpw)], wb0)
    pltpu.sync_copy(w1_hbm.at[pl.ds(base0, tpw)], wb1)

    incps = [None, None]
    sccps = {}
    incps[0] = pltpu.async_copy(
        xf_hbm.at[pl.ds(base0, sub)], xbufs[0], insems[0])
    for s in range(nsub):
        cur = s % 2
        if s >= 1:
            for cp in sccps[s - 1]:
                cp.wait()
        if s + 1 < nsub:
            incps[1 - cur] = pltpu.async_copy(
                xf_hbm.at[pl.ds(base0 + (s + 1) * sub, sub)],
                xbufs[1 - cur], insems[1 - cur])
        incps[cur].wait()
        i0 = idx0.at[s]
        i1 = idx1.at[s]
        sccps[s] = [
            pltpu.async_copy(xbufs[cur], ein_hbm.at[i0], scsems[cur]),
            pltpu.async_copy(xbufs[cur], ein_hbm.at[i1], scsems[cur]),
            pltpu.async_copy(wb0.at[pl.ds(s * sub, sub)], ws_hbm.at[i0],
                             scsems[cur]),
            pltpu.async_copy(wb1.at[pl.ds(s * sub, sub)], ws_hbm.at[i1],
                             scsems[cur]),
        ]
    for cp in sccps[nsub - 1]:
        cp.wait()


def _combine_body(T, D, sub, nsub, eout_hbm, c0_hbm, c1_hbm, y_hbm,
                  r0a, r0b, r1a, r1b, yba, ybb, idx0, idx1,
                  gsema, gsemb, wsema, wsemb):
    wid = lax.axis_index("s") * _NC + lax.axis_index("c")
    tpw = T // _NW
    base0 = wid * tpw
    r0s = (r0a, r0b)
    r1s = (r1a, r1b)
    ybs = (yba, ybb)
    gsems = (gsema, gsemb)
    wsems = (wsema, wsemb)

    pltpu.sync_copy(c0_hbm.at[pl.ds(base0, tpw)], idx0)
    pltpu.sync_copy(c1_hbm.at[pl.ds(base0, tpw)], idx1)

    def start_gathers(s, par):
        i0 = idx0.at[pl.ds(s * sub, sub)]
        i1 = idx1.at[pl.ds(s * sub, sub)]
        return [
            pltpu.async_copy(eout_hbm.at[i0], r0s[par], gsems[par]),
            pltpu.async_copy(eout_hbm.at[i1], r1s[par], gsems[par]),
        ]

    gcps = {0: start_gathers(0, 0)}
    wcps = {}
    for s in range(nsub):
        cur = s % 2
        if s + 1 < nsub:
            gcps[s + 1] = start_gathers(s + 1, 1 - cur)
        for cp in gcps[s]:
            cp.wait()
        if s >= 2:
            wcps[s - 2].wait()

        def tok_body(i, tc, cur=cur):
            for off in range(0, D, _L):
                ybs[cur][i, pl.ds(off, _L)] = (
                    r0s[cur][i, pl.ds(off, _L)] + r1s[cur][i, pl.ds(off, _L)])
            return tc

        lax.fori_loop(0, sub, tok_body, 0)
        wcps[s] = pltpu.async_copy(
            ybs[cur], y_hbm.at[pl.ds(base0 + s * sub, sub)], wsems[cur])
    wcps[nsub - 2].wait()
    wcps[nsub - 1].wait()


def kernel(x, Wg, W1, b1, W2, b2):
    Bx, Sx, D = x.shape
    T = Bx * Sx
    E = Wg.shape[1]
    H = W1.shape[2]
    cap = int(np.ceil(T * _K / E * _CAPF))
    nt = cap // _TILE
    xf = x.reshape(T, D)

    router = pl.pallas_call(
        functools.partial(_router_body, cap, T, E),
        out_shape=(
            jax.ShapeDtypeStruct((T, E), jnp.int32),
            jax.ShapeDtypeStruct((T, E), jnp.int32),
            jax.ShapeDtypeStruct((T, _WROW), jnp.float32),
            jax.ShapeDtypeStruct((T, _WROW), jnp.float32),
            jax.ShapeDtypeStruct((1, E), jnp.int32),
        ),
    )
    ds0, ds1, w0b, w1b, counts = router(xf, Wg)
    ds0f = ds0[:, 0]
    ds1f = ds1[:, 0]

    sub_d = 32
    nsub_d = (T // _NW) // sub_d
    mesh = plsc.VectorSubcoreMesh(
        core_axis_name="c", subcore_axis_name="s",
        num_cores=_NC, num_subcores=_NS)
    dispatch = functools.partial(
        pl.kernel,
        functools.partial(_dispatch_body, T, D, sub_d, nsub_d),
        out_type=(
            jax.ShapeDtypeStruct((E * cap + _TILE, D), jnp.float32),
            jax.ShapeDtypeStruct((E * cap + _TILE, _WROW), jnp.float32),
        ),
        mesh=mesh,
        scratch_types=[
            pltpu.VMEM((sub_d, D), jnp.float32),
            pltpu.VMEM((sub_d, D), jnp.float32),
            pltpu.VMEM((nsub_d, sub_d), jnp.int32),
            pltpu.VMEM((nsub_d, sub_d), jnp.int32),
            pltpu.VMEM((T // _NW, _WROW), jnp.float32),
            pltpu.VMEM((T // _NW, _WROW), jnp.float32),
            pltpu.SemaphoreType.DMA,
            pltpu.SemaphoreType.DMA,
            pltpu.SemaphoreType.DMA,
            pltpu.SemaphoreType.DMA,
        ],
    )()
    ein, wslot = dispatch(xf, ds0f.reshape(T // sub_d, sub_d),
                          ds1f.reshape(T // sub_d, sub_d), w0b, w1b)

    ew = lambda i: (jnp.minimum(i // nt, E - 1), 0, 0)
    expert = pl.pallas_call(
        functools.partial(_expert_body, cap, nt, E),
        grid=(E * nt + 1,),
        in_specs=[
            pl.BlockSpec(memory_space=pltpu.SMEM),
            pl.BlockSpec((_TILE, D), lambda i: (i, 0)),
            pl.BlockSpec((_TILE, _WROW), lambda i: (i, 0)),
            pl.BlockSpec((1, D, H), ew),
            pl.BlockSpec((1, 1, H), ew),
            pl.BlockSpec((1, H, D), ew),
            pl.BlockSpec((1, 1, D), ew),
        ],
        out_specs=pl.BlockSpec((_TILE, D), lambda i: (i, 0)),
        out_shape=jax.ShapeDtypeStruct((E * cap + _TILE, D), jnp.float32),
    )
    eout = expert(counts, ein, wslot, W1, b1.reshape(E, 1, H), W2,
                  b2.reshape(E, 1, D))

    sub_c = 16
    nsub_c = (T // _NW) // sub_c
    combine = functools.partial(
        pl.kernel,
        functools.partial(_combine_body, T, D, sub_c, nsub_c),
        out_type=jax.ShapeDtypeStruct((T, D), jnp.float32),
        mesh=mesh,
        scratch_types=[
            pltpu.VMEM((sub_c, D), jnp.float32),
            pltpu.VMEM((sub_c, D), jnp.float32),
            pltpu.VMEM((sub_c, D), jnp.float32),
            pltpu.VMEM((sub_c, D), jnp.float32),
            pltpu.VMEM((sub_c, D), jnp.float32),
            pltpu.VMEM((sub_c, D), jnp.float32),
            pltpu.VMEM((T // _NW,), jnp.int32),
            pltpu.VMEM((T // _NW,), jnp.int32),
            pltpu.SemaphoreType.DMA,
            pltpu.SemaphoreType.DMA,
            pltpu.SemaphoreType.DMA,
            pltpu.SemaphoreType.DMA,
        ],
    )()
    y = combine(eout, ds0f, ds1f)
    return y.reshape(Bx, Sx, D)
